# baseline (speedup 1.0000x reference)
import functools

import jax
import jax.numpy as jnp
from jax.experimental import pallas as pl
from jax.experimental.pallas import tpu as pltpu

_B = 4
_L = 8192
_H = 896
_H4 = 224
_K = 6
_BL = 1024


def _scores_body(b2_ref, h_ref, w1_ref, b1_ref, w2_ref, s_ref):
    h = h_ref[0]
    x = jnp.dot(h.astype(jnp.bfloat16), w1_ref[...].astype(jnp.bfloat16),
                preferred_element_type=jnp.float32)
    x = jnp.maximum(x + b1_ref[...], 0.0)
    xb = x.astype(jnp.bfloat16).astype(jnp.float32)
    w2b = w2_ref[...].astype(jnp.bfloat16).astype(jnp.float32)
    s = jnp.sum(xb * w2b, axis=1) + b2_ref[0]
    s_ref[...] = s.reshape(1, 1, _BL)


def _topk_body(s_ref, idx_ref, mask_ref):
    s = s_ref[...]
    col = jax.lax.broadcasted_iota(jnp.int32, (_B, _L), 1)
    lane = jax.lax.broadcasted_iota(jnp.int32, (_B, 128), 1)
    mask_acc = jnp.zeros((_B, _L), jnp.float32)
    idx_acc = jnp.zeros((_B, 128), jnp.int32)
    cur = s
    for k in range(_K):
        m = jnp.max(cur, axis=1, keepdims=True)
        idx = jnp.min(jnp.where(cur == m, col, _L), axis=1, keepdims=True)
        onehot = col == idx
        mask_acc = jnp.where(onehot, 1.0, mask_acc)
        cur = jnp.where(onehot, -jnp.inf, cur)
        idx_acc = jnp.where(lane == k, idx, idx_acc)
    mask_ref[...] = mask_acc
    idx_ref[...] = idx_acc


@jax.jit
def kernel(hidden_states, W1, b1, W2, b2):
    b, l, h = hidden_states.shape
    nb = b * l // _BL
    scores = pl.pallas_call(
        _scores_body,
        grid=(nb,),
        in_specs=[
            pl.BlockSpec(memory_space=pltpu.SMEM),
            pl.BlockSpec((1, _BL, _H), lambda i: (i, 0, 0)),
            pl.BlockSpec((_H, _H4), lambda i: (0, 0)),
            pl.BlockSpec((1, _H4), lambda i: (0, 0)),
            pl.BlockSpec((1, _H4), lambda i: (0, 0)),
        ],
        out_specs=pl.BlockSpec((1, 1, _BL), lambda i: (i, 0, 0)),
        out_shape=jax.ShapeDtypeStruct((nb, 1, _BL), jnp.float32),
    )(b2, hidden_states.reshape(nb, _BL, _H), W1,
      b1.reshape(1, _H4), W2.reshape(1, _H4))
    scores = scores.reshape(b, l)

    idx128, mask = pl.pallas_call(
        _topk_body,
        out_shape=(
            jax.ShapeDtypeStruct((b, 128), jnp.int32),
            jax.ShapeDtypeStruct((b, l), jnp.float32),
        ),
    )(scores)
    return scores, idx128[:, :_K], mask


# speedup vs baseline: 1.2565x; 1.2565x over previous
import functools

import jax
import jax.numpy as jnp
from jax.experimental import pallas as pl
from jax.experimental.pallas import tpu as pltpu

_B = 4
_L = 8192
_H = 896
_H4 = 224
_K = 6
_BL = 1024


def _scores_body(b2_ref, h_ref, w1_ref, b1_ref, w2_ref, s_ref):
    h = h_ref[0]
    x = jnp.dot(h, w1_ref[...], preferred_element_type=jnp.float32)
    x = jnp.maximum(x + b1_ref[...], 0.0)
    xb = x.astype(jnp.bfloat16)
    s = jax.lax.dot_general(w2_ref[...], xb, (((1,), (1,)), ((), ())),
                            preferred_element_type=jnp.float32)
    s_ref[...] = (s + b2_ref[0]).reshape(1, 1, _BL)


def _topk_body(s_ref, idx_ref, mask_ref):
    s = s_ref[...]
    col = jax.lax.broadcasted_iota(jnp.int32, (_B, _L), 1)
    lane = jax.lax.broadcasted_iota(jnp.int32, (_B, 128), 1)
    mask_acc = jnp.zeros((_B, _L), jnp.float32)
    idx_acc = jnp.zeros((_B, 128), jnp.int32)
    cur = s
    for k in range(_K):
        m = jnp.max(cur, axis=1, keepdims=True)
        idx = jnp.min(jnp.where(cur == m, col, _L), axis=1, keepdims=True)
        onehot = col == idx
        mask_acc = jnp.where(onehot, 1.0, mask_acc)
        cur = jnp.where(onehot, -jnp.inf, cur)
        idx_acc = jnp.where(lane == k, idx, idx_acc)
    mask_ref[...] = mask_acc
    idx_ref[...] = idx_acc


@jax.jit
def kernel(hidden_states, W1, b1, W2, b2):
    b, l, h = hidden_states.shape
    nb = b * l // _BL
    scores = pl.pallas_call(
        _scores_body,
        grid=(nb,),
        in_specs=[
            pl.BlockSpec(memory_space=pltpu.SMEM),
            pl.BlockSpec((1, _BL, _H), lambda i: (i, 0, 0)),
            pl.BlockSpec((_H, _H4), lambda i: (0, 0)),
            pl.BlockSpec((1, _H4), lambda i: (0, 0)),
            pl.BlockSpec((1, _H4), lambda i: (0, 0)),
        ],
        out_specs=pl.BlockSpec((1, 1, _BL), lambda i: (i, 0, 0)),
        out_shape=jax.ShapeDtypeStruct((nb, 1, _BL), jnp.float32),
    )(b2, hidden_states.reshape(nb, _BL, _H), W1.astype(jnp.bfloat16),
      b1.reshape(1, _H4), W2.reshape(1, _H4).astype(jnp.bfloat16))
    scores = scores.reshape(b, l)

    idx128, mask = pl.pallas_call(
        _topk_body,
        out_shape=(
            jax.ShapeDtypeStruct((b, 128), jnp.int32),
            jax.ShapeDtypeStruct((b, l), jnp.float32),
        ),
    )(scores)
    return scores, idx128[:, :_K], mask


# speedup vs baseline: 1.4889x; 1.1850x over previous
import functools

import jax
import jax.numpy as jnp
from jax.experimental import pallas as pl
from jax.experimental.pallas import tpu as pltpu

_B = 4
_L = 8192
_H = 896
_H4 = 224
_K = 6
_BL = 2048


def _scores_body(b2_ref, h_ref, w1_ref, b1_ref, w2_ref, s_ref):
    h = h_ref[0]
    x = jnp.dot(h, w1_ref[...], preferred_element_type=jnp.float32)
    x = jnp.maximum(x + b1_ref[...], 0.0)
    xb = x.astype(jnp.bfloat16)
    s = jax.lax.dot_general(w2_ref[...], xb, (((1,), (1,)), ((), ())),
                            preferred_element_type=jnp.float32)
    s_ref[...] = (s + b2_ref[0]).reshape(1, 1, _BL)


def _topk_body(s_ref, idx_ref, mask_ref):
    s = s_ref[...]
    col = jax.lax.broadcasted_iota(jnp.int32, (_B, _L), 1)
    lane = jax.lax.broadcasted_iota(jnp.int32, (_B, 128), 1)
    mask_acc = jnp.zeros((_B, _L), jnp.float32)
    idx_acc = jnp.zeros((_B, 128), jnp.int32)
    cur = s
    for k in range(_K):
        m = jnp.max(cur, axis=1, keepdims=True)
        idx = jnp.min(jnp.where(cur == m, col, _L), axis=1, keepdims=True)
        onehot = col == idx
        mask_acc = jnp.where(onehot, 1.0, mask_acc)
        cur = jnp.where(onehot, -jnp.inf, cur)
        idx_acc = jnp.where(lane == k, idx, idx_acc)
    mask_ref[...] = mask_acc
    idx_ref[...] = idx_acc


@jax.jit
def kernel(hidden_states, W1, b1, W2, b2):
    b, l, h = hidden_states.shape
    nb = b * l // _BL
    scores = pl.pallas_call(
        _scores_body,
        grid=(nb,),
        in_specs=[
            pl.BlockSpec(memory_space=pltpu.SMEM),
            pl.BlockSpec((1, _BL, _H), lambda i: (i, 0, 0)),
            pl.BlockSpec((_H, _H4), lambda i: (0, 0)),
            pl.BlockSpec((1, _H4), lambda i: (0, 0)),
            pl.BlockSpec((1, _H4), lambda i: (0, 0)),
        ],
        out_specs=pl.BlockSpec((1, 1, _BL), lambda i: (i, 0, 0)),
        out_shape=jax.ShapeDtypeStruct((nb, 1, _BL), jnp.float32),
        compiler_params=pltpu.CompilerParams(
            dimension_semantics=("parallel",)),
    )(b2, hidden_states.reshape(nb, _BL, _H), W1.astype(jnp.bfloat16),
      b1.reshape(1, _H4), W2.reshape(1, _H4).astype(jnp.bfloat16))
    scores = scores.reshape(b, l)

    idx128, mask = pl.pallas_call(
        _topk_body,
        out_shape=(
            jax.ShapeDtypeStruct((b, 128), jnp.int32),
            jax.ShapeDtypeStruct((b, l), jnp.float32),
        ),
    )(scores)
    return scores, idx128[:, :_K], mask


# speedup vs baseline: 1.5221x; 1.0223x over previous
import jax
import jax.numpy as jnp
from jax.experimental import pallas as pl
from jax.experimental.pallas import tpu as pltpu

_B = 4
_L = 8192
_H = 896
_H4 = 224
_K = 6
_BL = 2048
_NB = _B * _L // _BL
_JB = _L // _BL


def _body(b2_ref, h_ref, w1_ref, b1_ref, w2_ref,
          s_ref, idx_ref, mask_ref, acc_ref):
    i = pl.program_id(0)
    h = h_ref[0]
    x = jnp.dot(h, w1_ref[...], preferred_element_type=jnp.float32)
    x = jnp.maximum(x + b1_ref[...], 0.0)
    xb = x.astype(jnp.bfloat16)
    s = jax.lax.dot_general(w2_ref[...].astype(jnp.bfloat16), xb,
                            (((0,), (1,)), ((), ())),
                            preferred_element_type=jnp.float32)
    s = s + b2_ref[0]
    s_ref[...] = s.reshape(1, 1, _BL)
    acc_ref[pl.ds(i // _JB, 1), pl.ds((i % _JB) * _BL, _BL)] = s

    @pl.when(i == _NB - 1)
    def _epilogue():
        cur = acc_ref[...]
        col = jax.lax.broadcasted_iota(jnp.int32, (_B, _L), 1)
        lane = jax.lax.broadcasted_iota(jnp.int32, (_B, 128), 1)
        mask_acc = jnp.zeros((_B, _L), jnp.float32)
        idx_acc = jnp.zeros((_B, 128), jnp.int32)
        for k in range(_K):
            m = jnp.max(cur, axis=1, keepdims=True)
            idx = jnp.min(jnp.where(cur == m, col, _L), axis=1, keepdims=True)
            onehot = col == idx
            mask_acc = jnp.where(onehot, 1.0, mask_acc)
            cur = jnp.where(onehot, -jnp.inf, cur)
            idx_acc = jnp.where(lane == k, idx, idx_acc)
        mask_ref[...] = mask_acc
        idx_ref[...] = idx_acc


@jax.jit
def kernel(hidden_states, W1, b1, W2, b2):
    b, l, h = hidden_states.shape
    scores, idx128, mask = pl.pallas_call(
        _body,
        grid=(_NB,),
        in_specs=[
            pl.BlockSpec(memory_space=pltpu.SMEM),
            pl.BlockSpec((1, _BL, _H), lambda i: (i, 0, 0)),
            pl.BlockSpec((_H, _H4), lambda i: (0, 0)),
            pl.BlockSpec((1, _H4), lambda i: (0, 0)),
            pl.BlockSpec((_H4, 1), lambda i: (0, 0)),
        ],
        out_specs=(
            pl.BlockSpec((1, 1, _BL), lambda i: (i, 0, 0)),
            pl.BlockSpec((_B, 128), lambda i: (0, 0)),
            pl.BlockSpec((_B, _L), lambda i: (0, 0)),
        ),
        out_shape=(
            jax.ShapeDtypeStruct((_NB, 1, _BL), jnp.float32),
            jax.ShapeDtypeStruct((_B, 128), jnp.int32),
            jax.ShapeDtypeStruct((_B, _L), jnp.float32),
        ),
        scratch_shapes=[pltpu.VMEM((_B, _L), jnp.float32)],
        compiler_params=pltpu.CompilerParams(
            dimension_semantics=("arbitrary",)),
    )(b2, hidden_states.reshape(_NB, _BL, _H), W1.astype(jnp.bfloat16),
      b1.reshape(1, _H4), W2)
    return scores.reshape(b, l), idx128[:, :_K], mask
